# baseline (speedup 1.0000x reference)
import jax
import jax.numpy as jnp
from jax import lax
from jax.experimental import pallas as pl
from jax.experimental.pallas import tpu as pltpu
from jax.experimental.pallas import tpu_sc as plsc

BATCH = 4
SEQ = 2048
WIDTH = 512
NUM_CORES = 2
NUM_SUBCORES = 16
NUM_WORKERS = NUM_CORES * NUM_SUBCORES
S_PER_W = SEQ // NUM_WORKERS
ROWS = 16
NCHUNK = BATCH * S_PER_W // ROWS
HALVES = S_PER_W // ROWS
NBUF = 8
PRIME = 3
LANES = 16
CHUNKS = WIDTH // LANES

_LAG = NBUF - PRIME
_WAITED = set(c - _LAG for c in range(NCHUNK) if c + PRIME < NCHUNK and c >= _LAG)
_DRAIN = [c for c in range(NCHUNK) if c not in _WAITED]


def _sc_kernel(x_hbm, pos_hbm, table_hbm, out_hbm,
               idx_v, pos_v, bufs_v, gsem, ssem, psem):
    wid = lax.axis_index("s") * NUM_CORES + lax.axis_index("c")
    s_base = wid * S_PER_W

    idx_copies = [
        pltpu.async_copy(x_hbm.at[b, pl.ds(s_base, S_PER_W)],
                         idx_v.at[b], psem)
        for b in range(BATCH)
    ]
    for c in idx_copies:
        c.wait()

    def gather(c):
        h, b = divmod(c, BATCH)
        return pltpu.async_copy(
            table_hbm.at[idx_v.at[b, pl.ds(h * ROWS, ROWS)]],
            bufs_v.at[c % NBUF], gsem)

    gathers = [gather(0)]
    pos_copies = [pltpu.async_copy(
        pos_hbm.at[pl.ds(s_base + 0 * ROWS, ROWS)],
        pos_v.at[0], psem)]
    for k in range(1, PRIME):
        gathers.append(gather(k))
    for h in range(1, HALVES):
        pos_copies.append(pltpu.async_copy(
            pos_hbm.at[pl.ds(s_base + h * ROWS, ROWS)],
            pos_v.at[h], psem))

    stores = []
    for c in range(NCHUNK):
        h, b = divmod(c, BATCH)
        buf = bufs_v.at[c % NBUF]
        gathers[c].wait()
        if c + PRIME < NCHUNK:
            if c >= _LAG:
                stores[c - _LAG].wait()
            gathers.append(gather(c + PRIME))
        if b == 0:
            pos_copies[h].wait()

        def add_row(i, _, buf=buf, h=h):
            for j in range(CHUNKS):
                sl = pl.ds(j * LANES, LANES)
                buf[i, sl] = buf[i, sl] + pos_v[h, i, sl]
            return _

        lax.fori_loop(0, ROWS, add_row, None)
        stores.append(pltpu.async_copy(
            buf, out_hbm.at[b, pl.ds(s_base + h * ROWS, ROWS)], ssem))

    for c in _DRAIN:
        stores[c].wait()


@jax.jit
def _gpt_pre_encode(x, positional_embedding, token_embedding):
    mesh = plsc.VectorSubcoreMesh(core_axis_name="c", subcore_axis_name="s",
                                  num_cores=NUM_CORES,
                                  num_subcores=NUM_SUBCORES)
    run = pl.kernel(
        _sc_kernel,
        out_type=jax.ShapeDtypeStruct((BATCH, SEQ, WIDTH), jnp.float32),
        mesh=mesh,
        scratch_types=[
            pltpu.VMEM((BATCH, S_PER_W), jnp.int32),
            pltpu.VMEM((HALVES, ROWS, WIDTH), jnp.float32),
            pltpu.VMEM((NBUF, ROWS, WIDTH), jnp.float32),
            pltpu.SemaphoreType.DMA,
            pltpu.SemaphoreType.DMA,
            pltpu.SemaphoreType.DMA,
        ],
    )
    return run(x, positional_embedding, token_embedding)


def kernel(x, positional_embedding, token_embedding):
    return _gpt_pre_encode(x.astype(jnp.int32), positional_embedding,
                           token_embedding)


# speedup vs baseline: 1.1171x; 1.1171x over previous
import jax
import jax.numpy as jnp
from jax import lax
from jax.experimental import pallas as pl
from jax.experimental.pallas import tpu as pltpu
from jax.experimental.pallas import tpu_sc as plsc

BATCH = 4
SEQ = 2048
WIDTH = 512
NUM_CORES = 2
NUM_SUBCORES = 16
NUM_WORKERS = NUM_CORES * NUM_SUBCORES
S_PER_W = SEQ // NUM_WORKERS
ROWS = 32
NBUF = 5
PRIME = 2
LANES = 16
LCHUNK = WIDTH // LANES

_CHUNK_TBL = []
for _h in range(S_PER_W // ROWS):
    for _b in range(BATCH):
        if _h == 1 and _b == BATCH - 1:
            _CHUNK_TBL += [(_b, ROWS, ROWS // 2), (_b, ROWS + ROWS // 2, ROWS // 2)]
        else:
            _CHUNK_TBL.append((_b, _h * ROWS, ROWS))
_NCH = len(_CHUNK_TBL)


def _sc_kernel(x_hbm, pos_hbm, table_hbm, out_hbm,
               idx_v, pos_v, bufs_v, gsem, ssem, psem):
    wid = lax.axis_index("s") * NUM_CORES + lax.axis_index("c")
    s_base = wid * S_PER_W

    def gather(c):
        b, off, n = _CHUNK_TBL[c]
        return pltpu.async_copy(
            table_hbm.at[idx_v.at[b, pl.ds(off, n)]],
            bufs_v.at[pl.ds((c % NBUF) * ROWS, n)], gsem)

    idx_copies = [
        pltpu.async_copy(x_hbm.at[b, pl.ds(s_base, S_PER_W)],
                         idx_v.at[b], psem)
        for b in range(BATCH)
    ]
    idx_copies[0].wait()
    gathers = [gather(0)]
    for ic in idx_copies[1:]:
        ic.wait()
    pos_copies = [pltpu.async_copy(
        pos_hbm.at[pl.ds(s_base, ROWS)], pos_v.at[pl.ds(0, ROWS)], psem)]
    gathers.append(gather(1))
    pos_copies.append(pltpu.async_copy(
        pos_hbm.at[pl.ds(s_base + ROWS, ROWS)],
        pos_v.at[pl.ds(ROWS, ROWS)], psem))

    stores = []
    pos_waited = [False, False]
    for c in range(_NCH):
        b, off, n = _CHUNK_TBL[c]
        base = (c % NBUF) * ROWS
        gathers[c].wait()
        if c + PRIME < _NCH:
            if c + PRIME - NBUF >= 0:
                stores[c + PRIME - NBUF].wait()
            gathers.append(gather(c + PRIME))
        half = off // ROWS
        if not pos_waited[half]:
            pos_copies[half].wait()
            pos_waited[half] = True

        def add_row(i, _, base=base, off=off):
            for j in range(LCHUNK):
                sl = pl.ds(j * LANES, LANES)
                bufs_v[base + i, sl] = bufs_v[base + i, sl] + pos_v[off + i, sl]
            return _

        lax.fori_loop(0, n, add_row, None)
        stores.append(pltpu.async_copy(
            bufs_v.at[pl.ds(base, n)],
            out_hbm.at[b, pl.ds(s_base + off, n)], ssem))

    for c in range(_NCH - NBUF, _NCH):
        stores[c].wait()


@jax.jit
def _gpt_pre_encode(x, positional_embedding, token_embedding):
    mesh = plsc.VectorSubcoreMesh(core_axis_name="c", subcore_axis_name="s",
                                  num_cores=NUM_CORES,
                                  num_subcores=NUM_SUBCORES)
    run = pl.kernel(
        _sc_kernel,
        out_type=jax.ShapeDtypeStruct((BATCH, SEQ, WIDTH), jnp.float32),
        mesh=mesh,
        scratch_types=[
            pltpu.VMEM((BATCH, S_PER_W), jnp.int32),
            pltpu.VMEM((S_PER_W, WIDTH), jnp.float32),
            pltpu.VMEM((NBUF * ROWS, WIDTH), jnp.float32),
            pltpu.SemaphoreType.DMA,
            pltpu.SemaphoreType.DMA,
            pltpu.SemaphoreType.DMA,
        ],
    )
    return run(x, positional_embedding, token_embedding)


def kernel(x, positional_embedding, token_embedding):
    return _gpt_pre_encode(x.astype(jnp.int32), positional_embedding,
                           token_embedding)


# speedup vs baseline: 1.2329x; 1.1037x over previous
import jax
import jax.numpy as jnp
from jax import lax
from jax.experimental import pallas as pl
from jax.experimental.pallas import tpu as pltpu
from jax.experimental.pallas import tpu_sc as plsc

BATCH = 4
SEQ = 2048
WIDTH = 512
NUM_CORES = 2
NUM_SUBCORES = 16
NUM_WORKERS = NUM_CORES * NUM_SUBCORES
S_PER_W = SEQ // NUM_WORKERS
ROWS = 32
NCHUNK = BATCH * S_PER_W // ROWS
HALVES = S_PER_W // ROWS
NBUF = 4
LANES = 16
CHUNKS = WIDTH // LANES


def _sc_kernel(x_hbm, pos_hbm, table_hbm, out_hbm,
               idx_v, pos_v, buf0, buf1, buf2, buf3, gsem, ssem, psem):
    wid = lax.axis_index("s") * NUM_CORES + lax.axis_index("c")
    s_base = wid * S_PER_W

    idx_copies = [
        pltpu.async_copy(x_hbm.at[b, pl.ds(s_base, S_PER_W)],
                         idx_v.at[b], psem)
        for b in range(BATCH)
    ]
    bufs = (buf0, buf1, buf2, buf3)

    def gather(c):
        h, b = divmod(c, BATCH)
        return pltpu.async_copy(
            table_hbm.at[idx_v.at[b, pl.ds(h * ROWS, ROWS)]],
            bufs[c % NBUF], gsem)

    idx_copies[0].wait()
    gathers = [gather(0)]
    for c in idx_copies[1:]:
        c.wait()
    pos_copies = [pltpu.async_copy(
        pos_hbm.at[pl.ds(s_base + h * ROWS, ROWS)],
        pos_v.at[pl.ds(h * ROWS, ROWS)], psem) for h in range(HALVES)]
    gathers.append(gather(1))

    stores = []
    for c in range(NCHUNK):
        h, b = divmod(c, BATCH)
        buf = bufs[c % NBUF]
        gathers[c].wait()
        if c + 2 < NCHUNK:
            if c >= 2:
                stores[c - 2].wait()
            gathers.append(gather(c + 2))
        if c % BATCH == 0:
            pos_copies[h].wait()

        def add_row(i, _, buf=buf, h=h):
            for j in range(CHUNKS):
                sl = pl.ds(j * LANES, LANES)
                buf[i, sl] = buf[i, sl] + pos_v[h * ROWS + i, sl]
            return _

        lax.fori_loop(0, ROWS, add_row, None)
        stores.append(pltpu.async_copy(
            buf, out_hbm.at[b, pl.ds(s_base + h * ROWS, ROWS)], ssem))

    for c in range(NCHUNK - 4, NCHUNK):
        stores[c].wait()


@jax.jit
def _gpt_pre_encode(x, positional_embedding, token_embedding):
    mesh = plsc.VectorSubcoreMesh(core_axis_name="c", subcore_axis_name="s",
                                  num_cores=NUM_CORES,
                                  num_subcores=NUM_SUBCORES)
    run = pl.kernel(
        _sc_kernel,
        out_type=jax.ShapeDtypeStruct((BATCH, SEQ, WIDTH), jnp.float32),
        mesh=mesh,
        scratch_types=[
            pltpu.VMEM((BATCH, S_PER_W), jnp.int32),
            pltpu.VMEM((S_PER_W, WIDTH), jnp.float32),
            pltpu.VMEM((ROWS, WIDTH), jnp.float32),
            pltpu.VMEM((ROWS, WIDTH), jnp.float32),
            pltpu.VMEM((ROWS, WIDTH), jnp.float32),
            pltpu.VMEM((ROWS, WIDTH), jnp.float32),
            pltpu.SemaphoreType.DMA,
            pltpu.SemaphoreType.DMA,
            pltpu.SemaphoreType.DMA,
        ],
    )
    return run(x, positional_embedding, token_embedding)


def kernel(x, positional_embedding, token_embedding):
    return _gpt_pre_encode(x.astype(jnp.int32), positional_embedding,
                           token_embedding)
